# baseline (speedup 1.0000x reference)
import functools

import jax
import jax.numpy as jnp
from jax import lax
from jax.experimental import pallas as pl
from jax.experimental.pallas import tpu as pltpu
from jax.experimental.pallas import tpu_sc as plsc

K = 512
T_MAX = 100
EPS = 1e-6
B = 16
S = 2048

NB_SC = 8
NB_TC = B - NB_SC

NW = 32
TOK = NB_SC * S
TPW = TOK // NW
C = 64
NCHUNK = TPW // C

TS = 256


def _make_prep_body(b0):
    def body(tb1_ref, tb2_ref, t_ref, q1_ref, q2_ref, lt1_ref, lt2_ref):
        b = pl.program_id(0)
        tval = t_ref[b + b0]
        z = 1.0 + K * EPS

        q1 = q1_ref[0]
        lt1 = jnp.log(q1 + EPS)
        lt1 = jnp.where(tval == 1, jnp.zeros_like(lt1), lt1)
        lt1_ref[0] = lt1.astype(lt1_ref.dtype)

        q2 = q2_ref[0]
        colsum = jnp.sum(q2, axis=0, keepdims=True)
        lt2 = jnp.log(q2 + EPS * colsum + EPS * z) - jnp.log(z)
        row = lax.broadcasted_iota(jnp.int32, (K, K), 0)
        col = lax.broadcasted_iota(jnp.int32, (K, K), 1)
        eye_log = jnp.where(row == col, jnp.log(1.0 + EPS),
                            jnp.log(EPS)).astype(jnp.float32)
        lt2 = jnp.where(tval == 1, eye_log, lt2)
        lt2_ref[0] = lt2.astype(lt2_ref.dtype)
    return body


def _prep_tables(tb1, tb2, t, q_mats, q_one_step_transposed, b0, nb, dtype):
    grid_spec = pltpu.PrefetchScalarGridSpec(
        num_scalar_prefetch=3,
        grid=(nb,),
        in_specs=[
            pl.BlockSpec((1, K, K), lambda b, tb1, tb2, t: (tb1[b + b0], 0, 0)),
            pl.BlockSpec((1, K, K), lambda b, tb1, tb2, t: (tb2[b + b0], 0, 0)),
        ],
        out_specs=[
            pl.BlockSpec((1, K, K), lambda b, tb1, tb2, t: (b, 0, 0)),
            pl.BlockSpec((1, K, K), lambda b, tb1, tb2, t: (b, 0, 0)),
        ],
    )
    return pl.pallas_call(
        _make_prep_body(b0),
        grid_spec=grid_spec,
        out_shape=[
            jax.ShapeDtypeStruct((nb, K, K), dtype),
            jax.ShapeDtypeStruct((nb, K, K), dtype),
        ],
    )(tb1, tb2, t, q_one_step_transposed, q_mats)


@functools.lru_cache(maxsize=1)
def _make_gather_add():
    mesh = plsc.VectorSubcoreMesh(core_axis_name="c", subcore_axis_name="s")

    @functools.partial(
        pl.kernel,
        mesh=mesh,
        out_type=jax.ShapeDtypeStruct((B * S, K), jnp.float32),
        scratch_types=[
            pltpu.VMEM((C,), jnp.int32),
            pltpu.VMEM((C,), jnp.int32),
            pltpu.VMEM((C, K), jnp.float32),
            pltpu.VMEM((C, K), jnp.float32),
            pltpu.SemaphoreType.DMA,
            pltpu.SemaphoreType.DMA,
        ],
    )
    def _gather_add(lt1_hbm, lt2_hbm, gi1_hbm, gi2_hbm, out_hbm,
                    i1_v, i2_v, r1_v, r2_v, sem1, sem2):
        wid = lax.axis_index("s") * 2 + lax.axis_index("c")
        base = wid * TPW

        def chunk(ci, carry):
            tok = pl.multiple_of(base + ci * C, C)
            pltpu.sync_copy(gi1_hbm.at[pl.ds(tok, C)], i1_v)
            pltpu.sync_copy(gi2_hbm.at[pl.ds(tok, C)], i2_v)
            cp1 = pltpu.async_copy(lt1_hbm.at[i1_v], r1_v, sem1)
            cp2 = pltpu.async_copy(lt2_hbm.at[i2_v], r2_v, sem2)
            cp1.wait()
            cp2.wait()

            def addrow(r, carry2):
                for j in range(K // 16):
                    x = r2_v[r, pl.ds(j * 16, 16)]
                    plsc.addupdate(r1_v.at[r, pl.ds(j * 16, 16)], x)
                return carry2

            lax.fori_loop(0, C, addrow, 0)
            pltpu.sync_copy(r1_v, out_hbm.at[pl.ds(tok, C), :])
            return carry

        lax.fori_loop(0, NCHUNK, chunk, 0)

    return _gather_add


def _tc_body(tb1_ref, tb2_ref, t_ref, q1_ref, q2_ref, xt_ref, x0_ref,
             out_ref, lt1_s, lt2_s):
    b = pl.program_id(0)
    sb = pl.program_id(1)

    @pl.when(sb == 0)
    def _build_tables():
        tval = t_ref[b + NB_SC]
        z = 1.0 + K * EPS
        q1 = q1_ref[0]
        lt1 = jnp.log(q1 + EPS)
        lt1 = jnp.where(tval == 1, jnp.zeros_like(lt1), lt1)
        lt1_s[...] = lt1.astype(jnp.bfloat16)
        q2 = q2_ref[0]
        colsum = jnp.sum(q2, axis=0, keepdims=True)
        lt2 = jnp.log(q2 + EPS * colsum + EPS * z) - jnp.log(z)
        row = lax.broadcasted_iota(jnp.int32, (K, K), 0)
        col = lax.broadcasted_iota(jnp.int32, (K, K), 1)
        eye_log = jnp.where(row == col, jnp.log(1.0 + EPS),
                            jnp.log(EPS)).astype(jnp.float32)
        lt2 = jnp.where(tval == 1, eye_log, lt2)
        lt2_s[...] = lt2.astype(jnp.bfloat16)

    off = pl.multiple_of(sb * TS, TS)
    xt = xt_ref[0, 0, pl.ds(off, TS)]
    x0 = x0_ref[0, 0, pl.ds(off, TS)]
    cls = lax.broadcasted_iota(jnp.int32, (TS, K), 1)
    oh1 = (xt[:, None] == cls).astype(jnp.bfloat16)
    oh2 = (x0[:, None] == cls).astype(jnp.bfloat16)
    f1 = jnp.dot(oh1, lt1_s[...], preferred_element_type=jnp.float32)
    f2 = jnp.dot(oh2, lt2_s[...], preferred_element_type=jnp.float32)
    out_ref[0] = f1 + f2


def _tc_half(tb1, tb2, t, q_mats, q_one_step_transposed, xt_tc, x0_tc):
    grid_spec = pltpu.PrefetchScalarGridSpec(
        num_scalar_prefetch=3,
        grid=(NB_TC, S // TS),
        in_specs=[
            pl.BlockSpec((1, K, K), lambda b, sb, tb1, tb2, t: (tb1[b + NB_SC], 0, 0)),
            pl.BlockSpec((1, K, K), lambda b, sb, tb1, tb2, t: (tb2[b + NB_SC], 0, 0)),
            pl.BlockSpec((1, 1, S), lambda b, sb, tb1, tb2, t: (b, 0, 0)),
            pl.BlockSpec((1, 1, S), lambda b, sb, tb1, tb2, t: (b, 0, 0)),
        ],
        out_specs=pl.BlockSpec((1, TS, K), lambda b, sb, tb1, tb2, t: (b, sb, 0)),
        scratch_shapes=[
            pltpu.VMEM((K, K), jnp.bfloat16),
            pltpu.VMEM((K, K), jnp.bfloat16),
        ],
    )
    return pl.pallas_call(
        _tc_body,
        grid_spec=grid_spec,
        out_shape=jax.ShapeDtypeStruct((NB_TC, S, K), jnp.float32),
    )(tb1, tb2, t, q_one_step_transposed, q_mats, xt_tc, x0_tc)


def _join(out_full, out_tc):
    return pl.pallas_call(
        _join_body,
        grid=(NB_TC, S // TS),
        in_specs=[
            pl.BlockSpec(memory_space=pl.ANY),
            pl.BlockSpec((1, TS, K), lambda b, sb: (b, sb, 0)),
        ],
        out_specs=pl.BlockSpec((1, TS, K), lambda b, sb: (b + NB_SC, sb, 0)),
        out_shape=jax.ShapeDtypeStruct((B, S, K), jnp.float32),
        input_output_aliases={0: 0},
    )(out_full, out_tc)


def _join_body(full_ref, tc_ref, out_ref):
    out_ref[...] = tc_ref[...]


def kernel(x_0, x_t, t, q_mats, q_one_step_transposed):
    t = t.astype(jnp.int32)
    tb1 = (t - 1) % T_MAX
    tb2 = (t - 2) % T_MAX

    lt1, lt2 = _prep_tables(tb1, tb2, t, q_mats, q_one_step_transposed,
                            0, NB_SC, jnp.float32)
    boff = (jnp.arange(NB_SC, dtype=jnp.int32) * K)[:, None]
    gi1 = (x_t[:NB_SC].astype(jnp.int32) + boff).reshape(-1)
    gi2 = (x_0[:NB_SC].astype(jnp.int32) + boff).reshape(-1)
    out_sc = _make_gather_add()(
        lt1.reshape(NB_SC * K, K), lt2.reshape(NB_SC * K, K), gi1, gi2)

    xt_tc = x_t[NB_SC:].astype(jnp.int32).reshape(NB_TC, 1, S)
    x0_tc = x_0[NB_SC:].astype(jnp.int32).reshape(NB_TC, 1, S)
    out_tc = _tc_half(tb1, tb2, t, q_mats, q_one_step_transposed, xt_tc, x0_tc)

    return _join(out_sc.reshape(B, S, K), out_tc)


# speedup vs baseline: 1.0162x; 1.0162x over previous
import functools

import jax
import jax.numpy as jnp
from jax import lax
from jax.experimental import pallas as pl
from jax.experimental.pallas import tpu as pltpu
from jax.experimental.pallas import tpu_sc as plsc

K = 512
T_MAX = 100
EPS = 1e-6
B = 16
S = 2048

NB_SC = 8
NB_TC = B - NB_SC

NW = 32
TOK = NB_SC * S
TPW = TOK // NW
C = 64
NCHUNK = TPW // C

TS = 512


def _make_prep_body(b0):
    def body(tb1_ref, tb2_ref, t_ref, q1_ref, q2_ref, lt1_ref, lt2_ref):
        b = pl.program_id(0)
        tval = t_ref[b + b0]
        z = 1.0 + K * EPS

        q1 = q1_ref[0]
        lt1 = jnp.log(q1 + EPS)
        lt1 = jnp.where(tval == 1, jnp.zeros_like(lt1), lt1)
        lt1_ref[0] = lt1.astype(lt1_ref.dtype)

        q2 = q2_ref[0]
        colsum = jnp.sum(q2, axis=0, keepdims=True)
        lt2 = jnp.log(q2 + EPS * colsum + EPS * z) - jnp.log(z)
        row = lax.broadcasted_iota(jnp.int32, (K, K), 0)
        col = lax.broadcasted_iota(jnp.int32, (K, K), 1)
        eye_log = jnp.where(row == col, jnp.log(1.0 + EPS),
                            jnp.log(EPS)).astype(jnp.float32)
        lt2 = jnp.where(tval == 1, eye_log, lt2)
        lt2_ref[0] = lt2.astype(lt2_ref.dtype)
    return body


def _prep_tables(tb1, tb2, t, q_mats, q_one_step_transposed, b0, nb, dtype):
    grid_spec = pltpu.PrefetchScalarGridSpec(
        num_scalar_prefetch=3,
        grid=(nb,),
        in_specs=[
            pl.BlockSpec((1, K, K), lambda b, tb1, tb2, t: (tb1[b + b0], 0, 0)),
            pl.BlockSpec((1, K, K), lambda b, tb1, tb2, t: (tb2[b + b0], 0, 0)),
        ],
        out_specs=[
            pl.BlockSpec((1, K, K), lambda b, tb1, tb2, t: (b, 0, 0)),
            pl.BlockSpec((1, K, K), lambda b, tb1, tb2, t: (b, 0, 0)),
        ],
    )
    return pl.pallas_call(
        _make_prep_body(b0),
        grid_spec=grid_spec,
        out_shape=[
            jax.ShapeDtypeStruct((nb, K, K), dtype),
            jax.ShapeDtypeStruct((nb, K, K), dtype),
        ],
    )(tb1, tb2, t, q_one_step_transposed, q_mats)


@functools.lru_cache(maxsize=1)
def _make_gather_add():
    mesh = plsc.VectorSubcoreMesh(core_axis_name="c", subcore_axis_name="s")

    @functools.partial(
        pl.kernel,
        mesh=mesh,
        out_type=jax.ShapeDtypeStruct((TOK, K), jnp.float32),
        scratch_types=[
            pltpu.VMEM((C,), jnp.int32),
            pltpu.VMEM((C,), jnp.int32),
            pltpu.VMEM((C, K), jnp.float32),
            pltpu.VMEM((C, K), jnp.float32),
            pltpu.SemaphoreType.DMA,
            pltpu.SemaphoreType.DMA,
        ],
    )
    def _gather_add(lt1_hbm, lt2_hbm, gi1_hbm, gi2_hbm, out_hbm,
                    i1_v, i2_v, r1_v, r2_v, sem1, sem2):
        wid = lax.axis_index("s") * 2 + lax.axis_index("c")
        base = wid * TPW

        def chunk(ci, carry):
            tok = pl.multiple_of(base + ci * C, C)
            pltpu.sync_copy(gi1_hbm.at[pl.ds(tok, C)], i1_v)
            pltpu.sync_copy(gi2_hbm.at[pl.ds(tok, C)], i2_v)
            cp1 = pltpu.async_copy(lt1_hbm.at[i1_v], r1_v, sem1)
            cp2 = pltpu.async_copy(lt2_hbm.at[i2_v], r2_v, sem2)
            cp1.wait()
            cp2.wait()

            def addrow(r, carry2):
                for j in range(K // 16):
                    x = r2_v[r, pl.ds(j * 16, 16)]
                    plsc.addupdate(r1_v.at[r, pl.ds(j * 16, 16)], x)
                return carry2

            lax.fori_loop(0, C, addrow, 0)
            pltpu.sync_copy(r1_v, out_hbm.at[pl.ds(tok, C), :])
            return carry

        lax.fori_loop(0, NCHUNK, chunk, 0)

    return _gather_add


def _tc_body(tb1_ref, tb2_ref, t_ref, q1_ref, q2_ref, xt_ref, x0_ref,
             out_ref, lt1_s, lt2_s):
    b = pl.program_id(0)
    sb = pl.program_id(1)

    @pl.when(sb == 0)
    def _build_tables():
        tval = t_ref[b + NB_SC]
        z = 1.0 + K * EPS
        q1 = q1_ref[0]
        lt1 = jnp.log(q1 + EPS)
        lt1 = jnp.where(tval == 1, jnp.zeros_like(lt1), lt1)
        lt1_s[...] = lt1.astype(jnp.bfloat16)
        q2 = q2_ref[0]
        colsum = jnp.sum(q2, axis=0, keepdims=True)
        lt2 = jnp.log(q2 + EPS * colsum + EPS * z) - jnp.log(z)
        row = lax.broadcasted_iota(jnp.int32, (K, K), 0)
        col = lax.broadcasted_iota(jnp.int32, (K, K), 1)
        eye_log = jnp.where(row == col, jnp.log(1.0 + EPS),
                            jnp.log(EPS)).astype(jnp.float32)
        lt2 = jnp.where(tval == 1, eye_log, lt2)
        lt2_s[...] = lt2.astype(jnp.bfloat16)

    off = pl.multiple_of(sb * TS, TS)
    xt = xt_ref[0, 0, pl.ds(off, TS)]
    x0 = x0_ref[0, 0, pl.ds(off, TS)]
    cls = lax.broadcasted_iota(jnp.int32, (TS, K), 1)
    oh1 = (xt[:, None] == cls).astype(jnp.bfloat16)
    oh2 = (x0[:, None] == cls).astype(jnp.bfloat16)
    f1 = jnp.dot(oh1, lt1_s[...], preferred_element_type=jnp.float32)
    f2 = jnp.dot(oh2, lt2_s[...], preferred_element_type=jnp.float32)
    out_ref[0] = f1 + f2


def _tc_half(tb1, tb2, t, q_mats, q_one_step_transposed, xt_tc, x0_tc):
    grid_spec = pltpu.PrefetchScalarGridSpec(
        num_scalar_prefetch=3,
        grid=(NB_TC, S // TS),
        in_specs=[
            pl.BlockSpec((1, K, K), lambda b, sb, tb1, tb2, t: (tb1[b + NB_SC], 0, 0)),
            pl.BlockSpec((1, K, K), lambda b, sb, tb1, tb2, t: (tb2[b + NB_SC], 0, 0)),
            pl.BlockSpec((1, 1, S), lambda b, sb, tb1, tb2, t: (b, 0, 0)),
            pl.BlockSpec((1, 1, S), lambda b, sb, tb1, tb2, t: (b, 0, 0)),
        ],
        out_specs=pl.BlockSpec((1, TS, K), lambda b, sb, tb1, tb2, t: (b, sb, 0)),
        scratch_shapes=[
            pltpu.VMEM((K, K), jnp.bfloat16),
            pltpu.VMEM((K, K), jnp.bfloat16),
        ],
    )
    return pl.pallas_call(
        _tc_body,
        grid_spec=grid_spec,
        out_shape=jax.ShapeDtypeStruct((NB_TC, S, K), jnp.float32),
    )(tb1, tb2, t, q_one_step_transposed, q_mats, xt_tc, x0_tc)


def kernel(x_0, x_t, t, q_mats, q_one_step_transposed):
    t = t.astype(jnp.int32)
    tb1 = (t - 1) % T_MAX
    tb2 = (t - 2) % T_MAX

    lt1, lt2 = _prep_tables(tb1, tb2, t, q_mats, q_one_step_transposed,
                            0, NB_SC, jnp.float32)
    boff = (jnp.arange(NB_SC, dtype=jnp.int32) * K)[:, None]
    gi1 = (x_t[:NB_SC].astype(jnp.int32) + boff).reshape(-1)
    gi2 = (x_0[:NB_SC].astype(jnp.int32) + boff).reshape(-1)
    out_sc = _make_gather_add()(
        lt1.reshape(NB_SC * K, K), lt2.reshape(NB_SC * K, K), gi1, gi2)

    xt_tc = x_t[NB_SC:].astype(jnp.int32).reshape(NB_TC, 1, S)
    x0_tc = x_0[NB_SC:].astype(jnp.int32).reshape(NB_TC, 1, S)
    out_tc = _tc_half(tb1, tb2, t, q_mats, q_one_step_transposed, xt_tc, x0_tc)

    return jnp.concatenate([out_sc.reshape(NB_SC, S, K), out_tc], axis=0)


# speedup vs baseline: 1.0214x; 1.0052x over previous
import functools

import jax
import jax.numpy as jnp
from jax import lax
from jax.experimental import pallas as pl
from jax.experimental.pallas import tpu as pltpu
from jax.experimental.pallas import tpu_sc as plsc

K = 512
T_MAX = 100
EPS = 1e-6
B = 16
S = 2048

NB_SC = 7
NB_TC = B - NB_SC

NW = 32
TOK = NB_SC * S
TPW = TOK // NW
C = 64
NCHUNK = TPW // C

TS = 256


def _make_prep_body(b0):
    def body(tb1_ref, tb2_ref, t_ref, q1_ref, q2_ref, lt1_ref, lt2_ref):
        b = pl.program_id(0)
        tval = t_ref[b + b0]
        z = 1.0 + K * EPS

        q1 = q1_ref[0]
        lt1 = jnp.log(q1 + EPS)
        lt1 = jnp.where(tval == 1, jnp.zeros_like(lt1), lt1)
        lt1_ref[0] = lt1.astype(lt1_ref.dtype)

        q2 = q2_ref[0]
        colsum = jnp.sum(q2, axis=0, keepdims=True)
        lt2 = jnp.log(q2 + EPS * colsum + EPS * z) - jnp.log(z)
        row = lax.broadcasted_iota(jnp.int32, (K, K), 0)
        col = lax.broadcasted_iota(jnp.int32, (K, K), 1)
        eye_log = jnp.where(row == col, jnp.log(1.0 + EPS),
                            jnp.log(EPS)).astype(jnp.float32)
        lt2 = jnp.where(tval == 1, eye_log, lt2)
        lt2_ref[0] = lt2.astype(lt2_ref.dtype)
    return body


def _prep_tables(tb1, tb2, t, q_mats, q_one_step_transposed, b0, nb, dtype):
    grid_spec = pltpu.PrefetchScalarGridSpec(
        num_scalar_prefetch=3,
        grid=(nb,),
        in_specs=[
            pl.BlockSpec((1, K, K), lambda b, tb1, tb2, t: (tb1[b + b0], 0, 0)),
            pl.BlockSpec((1, K, K), lambda b, tb1, tb2, t: (tb2[b + b0], 0, 0)),
        ],
        out_specs=[
            pl.BlockSpec((1, K, K), lambda b, tb1, tb2, t: (b, 0, 0)),
            pl.BlockSpec((1, K, K), lambda b, tb1, tb2, t: (b, 0, 0)),
        ],
    )
    return pl.pallas_call(
        _make_prep_body(b0),
        grid_spec=grid_spec,
        out_shape=[
            jax.ShapeDtypeStruct((nb, K, K), dtype),
            jax.ShapeDtypeStruct((nb, K, K), dtype),
        ],
    )(tb1, tb2, t, q_one_step_transposed, q_mats)


@functools.lru_cache(maxsize=1)
def _make_gather_add():
    mesh = plsc.VectorSubcoreMesh(core_axis_name="c", subcore_axis_name="s")

    @functools.partial(
        pl.kernel,
        mesh=mesh,
        out_type=jax.ShapeDtypeStruct((TOK, K), jnp.float32),
        scratch_types=[
            pltpu.VMEM((C,), jnp.int32),
            pltpu.VMEM((C,), jnp.int32),
            pltpu.VMEM((C, K), jnp.float32),
            pltpu.VMEM((C, K), jnp.float32),
            pltpu.SemaphoreType.DMA,
            pltpu.SemaphoreType.DMA,
        ],
    )
    def _gather_add(lt1_hbm, lt2_hbm, gi1_hbm, gi2_hbm, out_hbm,
                    i1_v, i2_v, r1_v, r2_v, sem1, sem2):
        wid = lax.axis_index("s") * 2 + lax.axis_index("c")
        base = wid * TPW

        def chunk(ci, carry):
            tok = pl.multiple_of(base + ci * C, C)
            pltpu.sync_copy(gi1_hbm.at[pl.ds(tok, C)], i1_v)
            pltpu.sync_copy(gi2_hbm.at[pl.ds(tok, C)], i2_v)
            cp1 = pltpu.async_copy(lt1_hbm.at[i1_v], r1_v, sem1)
            cp2 = pltpu.async_copy(lt2_hbm.at[i2_v], r2_v, sem2)
            cp1.wait()
            cp2.wait()

            def addrow(r, carry2):
                for j in range(K // 16):
                    x = r2_v[r, pl.ds(j * 16, 16)]
                    plsc.addupdate(r1_v.at[r, pl.ds(j * 16, 16)], x)
                return carry2

            lax.fori_loop(0, C, addrow, 0)
            pltpu.sync_copy(r1_v, out_hbm.at[pl.ds(tok, C), :])
            return carry

        lax.fori_loop(0, NCHUNK, chunk, 0)

    return _gather_add


def _tc_body(tb1_ref, tb2_ref, t_ref, q1_ref, q2_ref, xt_ref, x0_ref,
             out_ref, lt1_s, lt2_s):
    b = pl.program_id(0)
    sb = pl.program_id(1)

    @pl.when(sb == 0)
    def _build_tables():
        tval = t_ref[b + NB_SC]
        z = 1.0 + K * EPS
        q1 = q1_ref[0]
        lt1 = jnp.log(q1 + EPS)
        lt1 = jnp.where(tval == 1, jnp.zeros_like(lt1), lt1)
        lt1_s[...] = lt1.astype(jnp.bfloat16)
        q2 = q2_ref[0]
        colsum = jnp.sum(q2, axis=0, keepdims=True)
        lt2 = jnp.log(q2 + EPS * colsum + EPS * z) - jnp.log(z)
        row = lax.broadcasted_iota(jnp.int32, (K, K), 0)
        col = lax.broadcasted_iota(jnp.int32, (K, K), 1)
        eye_log = jnp.where(row == col, jnp.log(1.0 + EPS),
                            jnp.log(EPS)).astype(jnp.float32)
        lt2 = jnp.where(tval == 1, eye_log, lt2)
        lt2_s[...] = lt2.astype(jnp.bfloat16)

    off = pl.multiple_of(sb * TS, TS)
    xt = xt_ref[0, 0, pl.ds(off, TS)]
    x0 = x0_ref[0, 0, pl.ds(off, TS)]
    cls = lax.broadcasted_iota(jnp.int32, (TS, K), 1)
    oh1 = (xt[:, None] == cls).astype(jnp.bfloat16)
    oh2 = (x0[:, None] == cls).astype(jnp.bfloat16)
    f1 = jnp.dot(oh1, lt1_s[...], preferred_element_type=jnp.float32)
    f2 = jnp.dot(oh2, lt2_s[...], preferred_element_type=jnp.float32)
    out_ref[0] = f1 + f2


def _tc_half(tb1, tb2, t, q_mats, q_one_step_transposed, xt_tc, x0_tc):
    grid_spec = pltpu.PrefetchScalarGridSpec(
        num_scalar_prefetch=3,
        grid=(NB_TC, S // TS),
        in_specs=[
            pl.BlockSpec((1, K, K), lambda b, sb, tb1, tb2, t: (tb1[b + NB_SC], 0, 0)),
            pl.BlockSpec((1, K, K), lambda b, sb, tb1, tb2, t: (tb2[b + NB_SC], 0, 0)),
            pl.BlockSpec((1, 1, S), lambda b, sb, tb1, tb2, t: (b, 0, 0)),
            pl.BlockSpec((1, 1, S), lambda b, sb, tb1, tb2, t: (b, 0, 0)),
        ],
        out_specs=pl.BlockSpec((1, TS, K), lambda b, sb, tb1, tb2, t: (b, sb, 0)),
        scratch_shapes=[
            pltpu.VMEM((K, K), jnp.bfloat16),
            pltpu.VMEM((K, K), jnp.bfloat16),
        ],
    )
    return pl.pallas_call(
        _tc_body,
        grid_spec=grid_spec,
        out_shape=jax.ShapeDtypeStruct((NB_TC, S, K), jnp.float32),
    )(tb1, tb2, t, q_one_step_transposed, q_mats, xt_tc, x0_tc)


def kernel(x_0, x_t, t, q_mats, q_one_step_transposed):
    t = t.astype(jnp.int32)
    tb1 = (t - 1) % T_MAX
    tb2 = (t - 2) % T_MAX

    lt1, lt2 = _prep_tables(tb1, tb2, t, q_mats, q_one_step_transposed,
                            0, NB_SC, jnp.float32)
    boff = (jnp.arange(NB_SC, dtype=jnp.int32) * K)[:, None]
    gi1 = (x_t[:NB_SC].astype(jnp.int32) + boff).reshape(-1)
    gi2 = (x_0[:NB_SC].astype(jnp.int32) + boff).reshape(-1)
    out_sc = _make_gather_add()(
        lt1.reshape(NB_SC * K, K), lt2.reshape(NB_SC * K, K), gi1, gi2)

    xt_tc = x_t[NB_SC:].astype(jnp.int32).reshape(NB_TC, 1, S)
    x0_tc = x_0[NB_SC:].astype(jnp.int32).reshape(NB_TC, 1, S)
    out_tc = _tc_half(tb1, tb2, t, q_mats, q_one_step_transposed, xt_tc, x0_tc)

    return jnp.concatenate([out_sc.reshape(NB_SC, S, K), out_tc], axis=0)


# speedup vs baseline: 1.0626x; 1.0403x over previous
import functools

import jax
import jax.numpy as jnp
from jax import lax
from jax.experimental import pallas as pl
from jax.experimental.pallas import tpu as pltpu
from jax.experimental.pallas import tpu_sc as plsc

K = 512
T_MAX = 100
EPS = 1e-6
B = 16
S = 2048

NB_SC = 8
NB_TC = B - NB_SC

NW = 32
TOK = NB_SC * S
TPW = TOK // NW
C = 32
NCHUNK = TPW // C

TS = 256


def _make_prep_body(b0):
    def body(tb1_ref, tb2_ref, t_ref, q1_ref, q2_ref, lt1_ref, lt2_ref):
        b = pl.program_id(0)
        tval = t_ref[b + b0]
        z = 1.0 + K * EPS

        q1 = q1_ref[0]
        lt1 = jnp.log(q1 + EPS)
        lt1 = jnp.where(tval == 1, jnp.zeros_like(lt1), lt1)
        lt1_ref[0] = lt1.astype(lt1_ref.dtype)

        q2 = q2_ref[0]
        colsum = jnp.sum(q2, axis=0, keepdims=True)
        lt2 = jnp.log(q2 + EPS * colsum + EPS * z) - jnp.log(z)
        row = lax.broadcasted_iota(jnp.int32, (K, K), 0)
        col = lax.broadcasted_iota(jnp.int32, (K, K), 1)
        eye_log = jnp.where(row == col, jnp.log(1.0 + EPS),
                            jnp.log(EPS)).astype(jnp.float32)
        lt2 = jnp.where(tval == 1, eye_log, lt2)
        lt2_ref[0] = lt2.astype(lt2_ref.dtype)
    return body


def _prep_tables(tb1, tb2, t, q_mats, q_one_step_transposed, b0, nb, dtype):
    grid_spec = pltpu.PrefetchScalarGridSpec(
        num_scalar_prefetch=3,
        grid=(nb,),
        in_specs=[
            pl.BlockSpec((1, K, K), lambda b, tb1, tb2, t: (tb1[b + b0], 0, 0)),
            pl.BlockSpec((1, K, K), lambda b, tb1, tb2, t: (tb2[b + b0], 0, 0)),
        ],
        out_specs=[
            pl.BlockSpec((1, K, K), lambda b, tb1, tb2, t: (b, 0, 0)),
            pl.BlockSpec((1, K, K), lambda b, tb1, tb2, t: (b, 0, 0)),
        ],
    )
    return pl.pallas_call(
        _make_prep_body(b0),
        grid_spec=grid_spec,
        out_shape=[
            jax.ShapeDtypeStruct((nb, K, K), dtype),
            jax.ShapeDtypeStruct((nb, K, K), dtype),
        ],
    )(tb1, tb2, t, q_one_step_transposed, q_mats)


@functools.lru_cache(maxsize=1)
def _make_gather_add():
    mesh = plsc.VectorSubcoreMesh(core_axis_name="c", subcore_axis_name="s")

    @functools.partial(
        pl.kernel,
        mesh=mesh,
        out_type=jax.ShapeDtypeStruct((TOK, K), jnp.float32),
        scratch_types=[
            pltpu.VMEM((2, C), jnp.int32),
            pltpu.VMEM((2, C), jnp.int32),
            pltpu.VMEM((2, C, K), jnp.float32),
            pltpu.VMEM((2, C, K), jnp.float32),
            pltpu.SemaphoreType.DMA,
            pltpu.SemaphoreType.DMA,
            pltpu.SemaphoreType.DMA,
            pltpu.SemaphoreType.DMA,
        ],
    )
    def _gather_add(lt1_hbm, lt2_hbm, gi1_hbm, gi2_hbm, out_hbm,
                    i1_v, i2_v, r1_v, r2_v, sg1a, sg1b, sg2a, sg2b):
        wid = lax.axis_index("s") * 2 + lax.axis_index("c")
        base = wid * TPW
        sg1 = (sg1a, sg1b)
        sg2 = (sg2a, sg2b)

        def issue(ci):
            s = ci % 2
            tok = pl.multiple_of(base + ci * C, C)
            pltpu.sync_copy(gi1_hbm.at[pl.ds(tok, C)], i1_v.at[s])
            pltpu.sync_copy(gi2_hbm.at[pl.ds(tok, C)], i2_v.at[s])
            c1 = pltpu.async_copy(lt1_hbm.at[i1_v.at[s]], r1_v.at[s], sg1[s])
            c2 = pltpu.async_copy(lt2_hbm.at[i2_v.at[s]], r2_v.at[s], sg2[s])
            return c1, c2

        pend = {0: issue(0)}
        for ci in range(NCHUNK):
            s = ci % 2
            if ci + 1 < NCHUNK:
                pend[ci + 1] = issue(ci + 1)
            c1, c2 = pend.pop(ci)
            c1.wait()
            c2.wait()

            def addrow(r, carry2, s=s):
                for j in range(K // 16):
                    x = r2_v[s, r, pl.ds(j * 16, 16)]
                    plsc.addupdate(r1_v.at[s, r, pl.ds(j * 16, 16)], x)
                return carry2

            lax.fori_loop(0, C, addrow, 0)
            tok = pl.multiple_of(base + ci * C, C)
            pltpu.sync_copy(r1_v.at[s], out_hbm.at[pl.ds(tok, C), :])

    return _gather_add


def _tc_body(tb1_ref, tb2_ref, t_ref, q1_ref, q2_ref, xt_ref, x0_ref,
             out_ref, lt1_s, lt2_s):
    b = pl.program_id(0)
    sb = pl.program_id(1)

    @pl.when(sb == 0)
    def _build_tables():
        tval = t_ref[b + NB_SC]
        z = 1.0 + K * EPS
        q1 = q1_ref[0]
        lt1 = jnp.log(q1 + EPS)
        lt1 = jnp.where(tval == 1, jnp.zeros_like(lt1), lt1)
        lt1_s[...] = lt1.astype(jnp.bfloat16)
        q2 = q2_ref[0]
        colsum = jnp.sum(q2, axis=0, keepdims=True)
        lt2 = jnp.log(q2 + EPS * colsum + EPS * z) - jnp.log(z)
        row = lax.broadcasted_iota(jnp.int32, (K, K), 0)
        col = lax.broadcasted_iota(jnp.int32, (K, K), 1)
        eye_log = jnp.where(row == col, jnp.log(1.0 + EPS),
                            jnp.log(EPS)).astype(jnp.float32)
        lt2 = jnp.where(tval == 1, eye_log, lt2)
        lt2_s[...] = lt2.astype(jnp.bfloat16)

    off = pl.multiple_of(sb * TS, TS)
    xt = xt_ref[0, 0, pl.ds(off, TS)]
    x0 = x0_ref[0, 0, pl.ds(off, TS)]
    cls = lax.broadcasted_iota(jnp.int32, (TS, K), 1)
    oh1 = (xt[:, None] == cls).astype(jnp.bfloat16)
    oh2 = (x0[:, None] == cls).astype(jnp.bfloat16)
    f1 = jnp.dot(oh1, lt1_s[...], preferred_element_type=jnp.float32)
    f2 = jnp.dot(oh2, lt2_s[...], preferred_element_type=jnp.float32)
    out_ref[0] = f1 + f2


def _tc_half(tb1, tb2, t, q_mats, q_one_step_transposed, xt_tc, x0_tc):
    grid_spec = pltpu.PrefetchScalarGridSpec(
        num_scalar_prefetch=3,
        grid=(NB_TC, S // TS),
        in_specs=[
            pl.BlockSpec((1, K, K), lambda b, sb, tb1, tb2, t: (tb1[b + NB_SC], 0, 0)),
            pl.BlockSpec((1, K, K), lambda b, sb, tb1, tb2, t: (tb2[b + NB_SC], 0, 0)),
            pl.BlockSpec((1, 1, S), lambda b, sb, tb1, tb2, t: (b, 0, 0)),
            pl.BlockSpec((1, 1, S), lambda b, sb, tb1, tb2, t: (b, 0, 0)),
        ],
        out_specs=pl.BlockSpec((1, TS, K), lambda b, sb, tb1, tb2, t: (b, sb, 0)),
        scratch_shapes=[
            pltpu.VMEM((K, K), jnp.bfloat16),
            pltpu.VMEM((K, K), jnp.bfloat16),
        ],
    )
    return pl.pallas_call(
        _tc_body,
        grid_spec=grid_spec,
        out_shape=jax.ShapeDtypeStruct((NB_TC, S, K), jnp.float32),
    )(tb1, tb2, t, q_one_step_transposed, q_mats, xt_tc, x0_tc)


def kernel(x_0, x_t, t, q_mats, q_one_step_transposed):
    t = t.astype(jnp.int32)
    tb1 = (t - 1) % T_MAX
    tb2 = (t - 2) % T_MAX

    lt1, lt2 = _prep_tables(tb1, tb2, t, q_mats, q_one_step_transposed,
                            0, NB_SC, jnp.float32)
    boff = (jnp.arange(NB_SC, dtype=jnp.int32) * K)[:, None]
    gi1 = (x_t[:NB_SC].astype(jnp.int32) + boff).reshape(-1)
    gi2 = (x_0[:NB_SC].astype(jnp.int32) + boff).reshape(-1)
    out_sc = _make_gather_add()(
        lt1.reshape(NB_SC * K, K), lt2.reshape(NB_SC * K, K), gi1, gi2)

    xt_tc = x_t[NB_SC:].astype(jnp.int32).reshape(NB_TC, 1, S)
    x0_tc = x_0[NB_SC:].astype(jnp.int32).reshape(NB_TC, 1, S)
    out_tc = _tc_half(tb1, tb2, t, q_mats, q_one_step_transposed, xt_tc, x0_tc)

    return jnp.concatenate([out_sc.reshape(NB_SC, S, K), out_tc], axis=0)
